# baseline (speedup 1.0000x reference)
import functools

import jax
import jax.numpy as jnp
from jax import lax
from jax.experimental import pallas as pl
from jax.experimental.pallas import tpu as pltpu
from jax.experimental.pallas import tpu_sc as plsc

N = 4096
BM = 256
K10 = 10
K5 = 5
INF = float("inf")

SC_NC = 2
SC_NW = 32
CHUNK = N // SC_NW
GROUPS = CHUNK // 16


def _dist(a_blk, bT):
    mm = jax.lax.dot_general(a_blk, bT, (((1,), (0,)), ((), ())),
                             preferred_element_type=jnp.float32)
    asq = jnp.sum(a_blk * a_blk, axis=1, keepdims=True)
    bsq = jnp.sum(bT * bT, axis=0, keepdims=True)
    return -2.0 * mm + asq + bsq


def _self1_body(src_ref, dstT_ref, gath_full_ref, gath_blk_ref, curv_ref):
    d = _dist(src_ref[...], dstT_ref[...])
    work = d
    for _ in range(K10):
        mv = jnp.min(work, axis=1, keepdims=True)
        work = jnp.where(work == mv, INF, work)
    onehot = (work == INF).astype(jnp.float32)
    gsum = jax.lax.dot_general(onehot, gath_full_ref[...],
                               (((1,), (0,)), ((), ())),
                               preferred_element_type=jnp.float32)
    curv_ref[...] = (gsum - 10.0 * gath_blk_ref[...]) / 9.0


def _self1_call(src, dstT, gath):
    return pl.pallas_call(
        _self1_body,
        grid=(N // BM,),
        in_specs=[
            pl.BlockSpec((BM, 3), lambda r: (r, 0)),
            pl.BlockSpec((3, N), lambda r: (0, 0)),
            pl.BlockSpec((N, 3), lambda r: (0, 0)),
            pl.BlockSpec((BM, 3), lambda r: (r, 0)),
        ],
        out_specs=pl.BlockSpec((BM, 3), lambda r: (r, 0)),
        out_shape=jax.ShapeDtypeStruct((N, 3), jnp.float32),
    )(src, dstT, gath, gath)


def _self2_body(src_ref, dstT_ref, gath_full_ref, gath_blk_ref,
                flowT_ref, flow_blk_ref, inter_blk_ref, sm_ref, cv_ref):
    r = pl.program_id(0)
    d = _dist(src_ref[...], dstT_ref[...])

    work = d
    sm_part = jnp.zeros((1, 1), jnp.float32)
    for t in range(K10):
        mv = jnp.min(work, axis=1, keepdims=True)
        work = jnp.where(work == mv, INF, work)
        if t == 8:
            m9 = work == INF
            dflow = _dist(flow_blk_ref[...], flowT_ref[...])
            nrm = jnp.sqrt(jnp.maximum(dflow, 0.0))
            sm_part = jnp.sum(jnp.where(m9, nrm, 0.0), keepdims=True)

    onehot = (work == INF).astype(jnp.float32)
    gsum = jax.lax.dot_general(onehot, gath_full_ref[...],
                               (((1,), (0,)), ((), ())),
                               preferred_element_type=jnp.float32)
    mc = (gsum - 10.0 * gath_blk_ref[...]) / 9.0
    cdiff = inter_blk_ref[...] - mc
    cv_part = jnp.sum(cdiff * cdiff, keepdims=True)

    @pl.when(r == 0)
    def _():
        sm_ref[...] = jnp.zeros((1, 1), jnp.float32)
        cv_ref[...] = jnp.zeros((1, 1), jnp.float32)
    sm_ref[...] += sm_part
    cv_ref[...] += cv_part


def _self2_call(src, dstT, gath, flowT, flow, inter):
    return pl.pallas_call(
        _self2_body,
        grid=(N // BM,),
        in_specs=[
            pl.BlockSpec((BM, 3), lambda r: (r, 0)),
            pl.BlockSpec((3, N), lambda r: (0, 0)),
            pl.BlockSpec((N, 3), lambda r: (0, 0)),
            pl.BlockSpec((BM, 3), lambda r: (r, 0)),
            pl.BlockSpec((3, N), lambda r: (0, 0)),
            pl.BlockSpec((BM, 3), lambda r: (r, 0)),
            pl.BlockSpec((BM, 3), lambda r: (r, 0)),
        ],
        out_specs=[
            pl.BlockSpec((1, 1), lambda r: (0, 0)),
            pl.BlockSpec((1, 1), lambda r: (0, 0)),
        ],
        out_shape=[
            jax.ShapeDtypeStruct((1, 1), jnp.float32),
            jax.ShapeDtypeStruct((1, 1), jnp.float32),
        ],
    )(src, dstT, gath, gath, flowT, flow, inter)


def _cross_body(src_ref, dstT_ref, dist2_ref, ch_ref, idx_ref, val_ref):
    r = pl.program_id(0)
    nr = pl.num_programs(0)
    d = _dist(src_ref[...], dstT_ref[...])

    colmin = jnp.min(d, axis=0, keepdims=True)

    iota = jax.lax.broadcasted_iota(jnp.int32, (BM, N), 1)
    lane16 = jax.lax.broadcasted_iota(jnp.int32, (BM, 16), 1)
    work = d
    idxacc = jnp.zeros((BM, 16), jnp.int32)
    valacc = jnp.zeros((BM, 16), jnp.float32)
    d1_part = jnp.zeros((1, 1), jnp.float32)
    for t in range(K5):
        mv = jnp.min(work, axis=1, keepdims=True)
        sel = jnp.min(jnp.where(work == mv, iota, N), axis=1, keepdims=True)
        idxacc = jnp.where(lane16 == t, sel, idxacc)
        valacc = jnp.where(lane16 == t, mv, valacc)
        if t == 0:
            d1_part = jnp.sum(mv, keepdims=True)
        work = jnp.where(work == mv, INF, work)

    idx_ref[...] = idxacc
    val_ref[...] = valacc

    @pl.when(r == 0)
    def _():
        dist2_ref[...] = colmin
        ch_ref[...] = jnp.zeros((1, 1), jnp.float32)

    @pl.when(r > 0)
    def _():
        dist2_ref[...] = jnp.minimum(dist2_ref[...], colmin)

    ch_ref[...] += d1_part

    @pl.when(r == nr - 1)
    def _():
        ch_ref[...] += jnp.sum(dist2_ref[...], keepdims=True)


def _cross_call(src, dstT):
    return pl.pallas_call(
        _cross_body,
        grid=(N // BM,),
        in_specs=[
            pl.BlockSpec((BM, 3), lambda r: (r, 0)),
            pl.BlockSpec((3, N), lambda r: (0, 0)),
        ],
        out_specs=[
            pl.BlockSpec((1, N), lambda r: (0, 0)),
            pl.BlockSpec((1, 1), lambda r: (0, 0)),
            pl.BlockSpec((BM, 16), lambda r: (r, 0)),
            pl.BlockSpec((BM, 16), lambda r: (r, 0)),
        ],
        out_shape=[
            jax.ShapeDtypeStruct((1, N), jnp.float32),
            jax.ShapeDtypeStruct((1, 1), jnp.float32),
            jax.ShapeDtypeStruct((N, 16), jnp.int32),
            jax.ShapeDtypeStruct((N, 16), jnp.float32),
        ],
    )(src, dstT)


def _sc_interp_body(c2_hbm, idx_hbm, val_hbm, out_hbm,
                    c2_v, idx_v, val_v, out_v):
    wid = lax.axis_index("s") * SC_NC + lax.axis_index("c")
    pltpu.sync_copy(c2_hbm, c2_v)
    pltpu.sync_copy(idx_hbm.at[pl.ds(wid * (CHUNK * 16), CHUNK * 16)], idx_v)
    pltpu.sync_copy(val_hbm.at[pl.ds(wid * (CHUNK * 16), CHUNK * 16)], val_v)
    iota = lax.iota(jnp.int32, 16)
    for g in range(GROUPS):
        lane_pt = g * 16 + iota
        nacc = jnp.zeros((16,), jnp.float32)
        ax = jnp.zeros((16,), jnp.float32)
        ay = jnp.zeros((16,), jnp.float32)
        az = jnp.zeros((16,), jnp.float32)
        for t in range(K5):
            pos = lane_pt * 16 + t
            iv = plsc.load_gather(idx_v, [pos])
            dv = plsc.load_gather(val_v, [pos])
            w = 1.0 / (dv + 1e-8)
            nacc = nacc + w
            iv3 = iv * 3
            ax = ax + w * plsc.load_gather(c2_v, [iv3])
            ay = ay + w * plsc.load_gather(c2_v, [iv3 + 1])
            az = az + w * plsc.load_gather(c2_v, [iv3 + 2])
        opos = lane_pt * 3
        plsc.store_scatter(out_v, [opos], ax / nacc)
        plsc.store_scatter(out_v, [opos + 1], ay / nacc)
        plsc.store_scatter(out_v, [opos + 2], az / nacc)
    pltpu.sync_copy(out_v, out_hbm.at[pl.ds(wid * (CHUNK * 3), CHUNK * 3)])


def _sc_interp_call(c2_flat, idx_flat, val_flat):
    mesh = plsc.VectorSubcoreMesh(core_axis_name="c", subcore_axis_name="s")
    fn = functools.partial(
        pl.kernel,
        mesh=mesh,
        compiler_params=pltpu.CompilerParams(needs_layout_passes=False),
        out_type=jax.ShapeDtypeStruct((N * 3,), jnp.float32),
        scratch_types=[
            pltpu.VMEM((N * 3,), jnp.float32),
            pltpu.VMEM((CHUNK * 16,), jnp.int32),
            pltpu.VMEM((CHUNK * 16,), jnp.float32),
            pltpu.VMEM((CHUNK * 3,), jnp.float32),
        ],
    )(_sc_interp_body)
    return fn(c2_flat, idx_flat, val_flat)


def kernel(registration_pred, registration_gt, coords):
    flow = registration_pred[0]
    pc1 = coords
    pc2 = coords + registration_gt[0]
    warp = pc1 + flow

    pc1T = pc1.T
    pc2T = pc2.T
    flowT = flow.T

    c2 = _self1_call(pc2, pc2T, pc2)
    _dist2, ch, idxw2, valw2 = _cross_call(warp, pc2T)
    inter = _sc_interp_call(c2.reshape(-1), idxw2.reshape(-1),
                            valw2.reshape(-1)).reshape(N, 3)
    sm, cv = _self2_call(pc1, pc1T, warp, flowT, flow, inter)

    chamfer = ch[0, 0]
    curv = cv[0, 0]
    smooth = sm[0, 0] / 8.0

    alpha = 0.02
    total = alpha * chamfer + 0.3 * (alpha * curv) + alpha * smooth
    return jnp.reshape(total, (1,))


# speedup vs baseline: 1.0015x; 1.0015x over previous
import functools

import jax
import jax.numpy as jnp
from jax import lax
from jax.experimental import pallas as pl
from jax.experimental.pallas import tpu as pltpu
from jax.experimental.pallas import tpu_sc as plsc

N = 4096
BM = 256
K10 = 10
K5 = 5
INF = float("inf")

SC_NC = 2
SC_NW = 32
CHUNK = N // SC_NW
GROUPS = CHUNK // 16


def _dist(a_blk, bT):
    mm = jax.lax.dot_general(a_blk, bT, (((1,), (0,)), ((), ())),
                             preferred_element_type=jnp.float32)
    asq = jnp.sum(a_blk * a_blk, axis=1, keepdims=True)
    bsq = jnp.sum(bT * bT, axis=0, keepdims=True)
    return -2.0 * mm + asq + bsq


def _self1_body(src_ref, dstT_ref, gath_full_ref, gath_blk_ref, curv_ref):
    d = _dist(src_ref[...], dstT_ref[...])
    work = d
    for _ in range(K10):
        mv = jnp.min(work, axis=1, keepdims=True)
        work = jnp.where(work == mv, INF, work)
    onehot = (work == INF).astype(jnp.float32)
    gsum = jax.lax.dot_general(onehot, gath_full_ref[...],
                               (((1,), (0,)), ((), ())),
                               preferred_element_type=jnp.float32)
    curv_ref[...] = (gsum - 10.0 * gath_blk_ref[...]) / 9.0


def _self1_call(src, dstT, gath):
    return pl.pallas_call(
        _self1_body,
        grid=(N // BM,),
        in_specs=[
            pl.BlockSpec((BM, 3), lambda r: (r, 0)),
            pl.BlockSpec((3, N), lambda r: (0, 0)),
            pl.BlockSpec((N, 3), lambda r: (0, 0)),
            pl.BlockSpec((BM, 3), lambda r: (r, 0)),
        ],
        out_specs=pl.BlockSpec((BM, 3), lambda r: (r, 0)),
        out_shape=jax.ShapeDtypeStruct((N, 3), jnp.float32),
    )(src, dstT, gath, gath)


def _self2_body(src_ref, dstT_ref, gath_full_ref, gath_blk_ref,
                flowT_ref, flow_blk_ref, mc_ref, sm_ref):
    r = pl.program_id(0)
    d = _dist(src_ref[...], dstT_ref[...])

    work = d
    sm_part = jnp.zeros((1, 1), jnp.float32)
    for t in range(K10):
        mv = jnp.min(work, axis=1, keepdims=True)
        work = jnp.where(work == mv, INF, work)
        if t == 8:
            m9 = work == INF
            dflow = _dist(flow_blk_ref[...], flowT_ref[...])
            nrm = jnp.sqrt(jnp.maximum(dflow, 0.0))
            sm_part = jnp.sum(jnp.where(m9, nrm, 0.0), keepdims=True)

    onehot = (work == INF).astype(jnp.float32)
    gsum = jax.lax.dot_general(onehot, gath_full_ref[...],
                               (((1,), (0,)), ((), ())),
                               preferred_element_type=jnp.float32)
    mc_ref[...] = (gsum - 10.0 * gath_blk_ref[...]) / 9.0

    @pl.when(r == 0)
    def _():
        sm_ref[...] = jnp.zeros((1, 1), jnp.float32)
    sm_ref[...] += sm_part


def _self2_call(src, dstT, gath, flowT, flow):
    return pl.pallas_call(
        _self2_body,
        grid=(N // BM,),
        in_specs=[
            pl.BlockSpec((BM, 3), lambda r: (r, 0)),
            pl.BlockSpec((3, N), lambda r: (0, 0)),
            pl.BlockSpec((N, 3), lambda r: (0, 0)),
            pl.BlockSpec((BM, 3), lambda r: (r, 0)),
            pl.BlockSpec((3, N), lambda r: (0, 0)),
            pl.BlockSpec((BM, 3), lambda r: (r, 0)),
        ],
        out_specs=[
            pl.BlockSpec((BM, 3), lambda r: (r, 0)),
            pl.BlockSpec((1, 1), lambda r: (0, 0)),
        ],
        out_shape=[
            jax.ShapeDtypeStruct((N, 3), jnp.float32),
            jax.ShapeDtypeStruct((1, 1), jnp.float32),
        ],
    )(src, dstT, gath, gath, flowT, flow)


def _cv_body(inter_ref, mc_ref, cv_ref):
    cdiff = inter_ref[...] - mc_ref[...]
    cv_ref[...] = jnp.sum(cdiff * cdiff, keepdims=True)


def _cv_call(inter, mc):
    return pl.pallas_call(
        _cv_body,
        out_shape=jax.ShapeDtypeStruct((1, 1), jnp.float32),
    )(inter, mc)


def _cross_body(src_ref, dstT_ref, dist2_ref, ch_ref, idx_ref, val_ref):
    r = pl.program_id(0)
    nr = pl.num_programs(0)
    d = _dist(src_ref[...], dstT_ref[...])

    colmin = jnp.min(d, axis=0, keepdims=True)

    iota = jax.lax.broadcasted_iota(jnp.int32, (BM, N), 1)
    lane16 = jax.lax.broadcasted_iota(jnp.int32, (BM, 16), 1)
    work = d
    idxacc = jnp.zeros((BM, 16), jnp.int32)
    valacc = jnp.zeros((BM, 16), jnp.float32)
    d1_part = jnp.zeros((1, 1), jnp.float32)
    for t in range(K5):
        mv = jnp.min(work, axis=1, keepdims=True)
        sel = jnp.min(jnp.where(work == mv, iota, N), axis=1, keepdims=True)
        idxacc = jnp.where(lane16 == t, sel, idxacc)
        valacc = jnp.where(lane16 == t, mv, valacc)
        if t == 0:
            d1_part = jnp.sum(mv, keepdims=True)
        work = jnp.where(work == mv, INF, work)

    idx_ref[...] = idxacc
    val_ref[...] = valacc

    @pl.when(r == 0)
    def _():
        dist2_ref[...] = colmin
        ch_ref[...] = jnp.zeros((1, 1), jnp.float32)

    @pl.when(r > 0)
    def _():
        dist2_ref[...] = jnp.minimum(dist2_ref[...], colmin)

    ch_ref[...] += d1_part

    @pl.when(r == nr - 1)
    def _():
        ch_ref[...] += jnp.sum(dist2_ref[...], keepdims=True)


def _cross_call(src, dstT):
    return pl.pallas_call(
        _cross_body,
        grid=(N // BM,),
        in_specs=[
            pl.BlockSpec((BM, 3), lambda r: (r, 0)),
            pl.BlockSpec((3, N), lambda r: (0, 0)),
        ],
        out_specs=[
            pl.BlockSpec((1, N), lambda r: (0, 0)),
            pl.BlockSpec((1, 1), lambda r: (0, 0)),
            pl.BlockSpec((BM, 16), lambda r: (r, 0)),
            pl.BlockSpec((BM, 16), lambda r: (r, 0)),
        ],
        out_shape=[
            jax.ShapeDtypeStruct((1, N), jnp.float32),
            jax.ShapeDtypeStruct((1, 1), jnp.float32),
            jax.ShapeDtypeStruct((N, 16), jnp.int32),
            jax.ShapeDtypeStruct((N, 16), jnp.float32),
        ],
    )(src, dstT)


def _sc_interp_body(c2_hbm, idx_hbm, val_hbm, out_hbm,
                    c2_v, idx_v, val_v, out_v):
    wid = lax.axis_index("s") * SC_NC + lax.axis_index("c")
    pltpu.sync_copy(c2_hbm, c2_v)
    pltpu.sync_copy(idx_hbm.at[pl.ds(wid * (CHUNK * 16), CHUNK * 16)], idx_v)
    pltpu.sync_copy(val_hbm.at[pl.ds(wid * (CHUNK * 16), CHUNK * 16)], val_v)
    iota = lax.iota(jnp.int32, 16)
    for g in range(GROUPS):
        lane_pt = g * 16 + iota
        nacc = jnp.zeros((16,), jnp.float32)
        ax = jnp.zeros((16,), jnp.float32)
        ay = jnp.zeros((16,), jnp.float32)
        az = jnp.zeros((16,), jnp.float32)
        for t in range(K5):
            pos = lane_pt * 16 + t
            iv = plsc.load_gather(idx_v, [pos])
            dv = plsc.load_gather(val_v, [pos])
            w = 1.0 / (dv + 1e-8)
            nacc = nacc + w
            iv3 = iv * 3
            ax = ax + w * plsc.load_gather(c2_v, [iv3])
            ay = ay + w * plsc.load_gather(c2_v, [iv3 + 1])
            az = az + w * plsc.load_gather(c2_v, [iv3 + 2])
        opos = lane_pt * 3
        plsc.store_scatter(out_v, [opos], ax / nacc)
        plsc.store_scatter(out_v, [opos + 1], ay / nacc)
        plsc.store_scatter(out_v, [opos + 2], az / nacc)
    pltpu.sync_copy(out_v, out_hbm.at[pl.ds(wid * (CHUNK * 3), CHUNK * 3)])


def _sc_interp_call(c2_flat, idx_flat, val_flat):
    mesh = plsc.VectorSubcoreMesh(core_axis_name="c", subcore_axis_name="s")
    fn = functools.partial(
        pl.kernel,
        mesh=mesh,
        compiler_params=pltpu.CompilerParams(needs_layout_passes=False),
        out_type=jax.ShapeDtypeStruct((N * 3,), jnp.float32),
        scratch_types=[
            pltpu.VMEM((N * 3,), jnp.float32),
            pltpu.VMEM((CHUNK * 16,), jnp.int32),
            pltpu.VMEM((CHUNK * 16,), jnp.float32),
            pltpu.VMEM((CHUNK * 3,), jnp.float32),
        ],
    )(_sc_interp_body)
    return fn(c2_flat, idx_flat, val_flat)


def kernel(registration_pred, registration_gt, coords):
    flow = registration_pred[0]
    pc1 = coords
    pc2 = coords + registration_gt[0]
    warp = pc1 + flow

    pc1T = pc1.T
    pc2T = pc2.T
    flowT = flow.T

    c2 = _self1_call(pc2, pc2T, pc2)
    _dist2, ch, idxw2, valw2 = _cross_call(warp, pc2T)
    inter = _sc_interp_call(c2.reshape(-1), idxw2.reshape(-1),
                            valw2.reshape(-1)).reshape(N, 3)
    mc, sm = _self2_call(pc1, pc1T, warp, flowT, flow)
    cv = _cv_call(inter, mc)

    chamfer = ch[0, 0]
    curv = cv[0, 0]
    smooth = sm[0, 0] / 8.0

    alpha = 0.02
    total = alpha * chamfer + 0.3 * (alpha * curv) + alpha * smooth
    return jnp.reshape(total, (1,))


# speedup vs baseline: 1.3364x; 1.3344x over previous
import functools

import jax
import jax.numpy as jnp
from jax import lax
from jax.experimental import pallas as pl
from jax.experimental.pallas import tpu as pltpu
from jax.experimental.pallas import tpu_sc as plsc

N = 4096
BM = 256
K10 = 10
K5 = 5
INF = float("inf")
LANES = 128
CHUNKS = N // LANES

SC_NC = 2
SC_NW = 32
CHUNK = N // SC_NW
GROUPS = CHUNK // 16


def _dist(a_blk, bT):
    mm = jax.lax.dot_general(a_blk, bT, (((1,), (0,)), ((), ())),
                             preferred_element_type=jnp.float32)
    asq = jnp.sum(a_blk * a_blk, axis=1, keepdims=True)
    bsq = jnp.sum(bT * bT, axis=0, keepdims=True)
    return -2.0 * mm + asq + bsq


def _topk_vals(d, k, bm):
    m1 = jnp.full((bm, LANES), INF, jnp.float32)
    m2 = jnp.full((bm, LANES), INF, jnp.float32)
    m3 = jnp.full((bm, LANES), INF, jnp.float32)
    m4 = jnp.full((bm, LANES), INF, jnp.float32)
    for c in range(CHUNKS):
        x = d[:, LANES * c:LANES * (c + 1)]
        t1 = jnp.minimum(m1, x)
        x = jnp.maximum(m1, x)
        m1 = t1
        t2 = jnp.minimum(m2, x)
        x = jnp.maximum(m2, x)
        m2 = t2
        t3 = jnp.minimum(m3, x)
        x = jnp.maximum(m3, x)
        m3 = t3
        m4 = jnp.minimum(m4, x)
    vals = []
    for _ in range(k):
        vt = jnp.min(m1, axis=1, keepdims=True)
        hit = m1 == vt
        m1 = jnp.where(hit, m2, m1)
        m2 = jnp.where(hit, m3, m2)
        m3 = jnp.where(hit, m4, m3)
        m4 = jnp.where(hit, INF, m4)
        vals.append(vt)
    return vals


def _self1_body(src_ref, dstT_ref, gath_full_ref, gath_blk_ref, curv_ref):
    d = _dist(src_ref[...], dstT_ref[...])
    vals = _topk_vals(d, K10, BM)
    onehot = (d <= vals[9]).astype(jnp.float32)
    gsum = jax.lax.dot_general(onehot, gath_full_ref[...],
                               (((1,), (0,)), ((), ())),
                               preferred_element_type=jnp.float32)
    curv_ref[...] = (gsum - 10.0 * gath_blk_ref[...]) / 9.0


def _self1_call(src, dstT, gath):
    return pl.pallas_call(
        _self1_body,
        grid=(N // BM,),
        in_specs=[
            pl.BlockSpec((BM, 3), lambda r: (r, 0)),
            pl.BlockSpec((3, N), lambda r: (0, 0)),
            pl.BlockSpec((N, 3), lambda r: (0, 0)),
            pl.BlockSpec((BM, 3), lambda r: (r, 0)),
        ],
        out_specs=pl.BlockSpec((BM, 3), lambda r: (r, 0)),
        out_shape=jax.ShapeDtypeStruct((N, 3), jnp.float32),
    )(src, dstT, gath, gath)


def _self2_body(src_ref, dstT_ref, gath_full_ref, gath_blk_ref,
                flowT_ref, flow_blk_ref, mc_ref, sm_ref):
    r = pl.program_id(0)
    d = _dist(src_ref[...], dstT_ref[...])
    vals = _topk_vals(d, K10, BM)

    m9 = d <= vals[8]
    dflow = _dist(flow_blk_ref[...], flowT_ref[...])
    nrm = jnp.sqrt(jnp.maximum(dflow, 0.0))
    sm_part = jnp.sum(jnp.where(m9, nrm, 0.0), keepdims=True)

    onehot = (d <= vals[9]).astype(jnp.float32)
    gsum = jax.lax.dot_general(onehot, gath_full_ref[...],
                               (((1,), (0,)), ((), ())),
                               preferred_element_type=jnp.float32)
    mc_ref[...] = (gsum - 10.0 * gath_blk_ref[...]) / 9.0

    @pl.when(r == 0)
    def _():
        sm_ref[...] = jnp.zeros((1, 1), jnp.float32)
    sm_ref[...] += sm_part


def _self2_call(src, dstT, gath, flowT, flow):
    return pl.pallas_call(
        _self2_body,
        grid=(N // BM,),
        in_specs=[
            pl.BlockSpec((BM, 3), lambda r: (r, 0)),
            pl.BlockSpec((3, N), lambda r: (0, 0)),
            pl.BlockSpec((N, 3), lambda r: (0, 0)),
            pl.BlockSpec((BM, 3), lambda r: (r, 0)),
            pl.BlockSpec((3, N), lambda r: (0, 0)),
            pl.BlockSpec((BM, 3), lambda r: (r, 0)),
        ],
        out_specs=[
            pl.BlockSpec((BM, 3), lambda r: (r, 0)),
            pl.BlockSpec((1, 1), lambda r: (0, 0)),
        ],
        out_shape=[
            jax.ShapeDtypeStruct((N, 3), jnp.float32),
            jax.ShapeDtypeStruct((1, 1), jnp.float32),
        ],
    )(src, dstT, gath, gath, flowT, flow)


def _cv_body(inter_ref, mc_ref, cv_ref):
    cdiff = inter_ref[...] - mc_ref[...]
    cv_ref[...] = jnp.sum(cdiff * cdiff, keepdims=True)


def _cv_call(inter, mc):
    return pl.pallas_call(
        _cv_body,
        out_shape=jax.ShapeDtypeStruct((1, 1), jnp.float32),
    )(inter, mc)


def _cross_body(src_ref, dstT_ref, dist2_ref, ch_ref, idx_ref, val_ref):
    r = pl.program_id(0)
    nr = pl.num_programs(0)
    d = _dist(src_ref[...], dstT_ref[...])

    colmin = jnp.min(d, axis=0, keepdims=True)
    vals = _topk_vals(d, K5, BM)
    d1_part = jnp.sum(vals[0], keepdims=True)

    iota = jax.lax.broadcasted_iota(jnp.int32, (BM, N), 1)
    lane16 = jax.lax.broadcasted_iota(jnp.int32, (BM, 16), 1)
    idxacc = jnp.zeros((BM, 16), jnp.int32)
    valacc = jnp.zeros((BM, 16), jnp.float32)
    for t in range(K5):
        sel = jnp.min(jnp.where(d == vals[t], iota, N), axis=1, keepdims=True)
        idxacc = jnp.where(lane16 == t, sel, idxacc)
        valacc = jnp.where(lane16 == t, vals[t], valacc)

    idx_ref[...] = idxacc
    val_ref[...] = valacc

    @pl.when(r == 0)
    def _():
        dist2_ref[...] = colmin
        ch_ref[...] = jnp.zeros((1, 1), jnp.float32)

    @pl.when(r > 0)
    def _():
        dist2_ref[...] = jnp.minimum(dist2_ref[...], colmin)

    ch_ref[...] += d1_part

    @pl.when(r == nr - 1)
    def _():
        ch_ref[...] += jnp.sum(dist2_ref[...], keepdims=True)


def _cross_call(src, dstT):
    return pl.pallas_call(
        _cross_body,
        grid=(N // BM,),
        in_specs=[
            pl.BlockSpec((BM, 3), lambda r: (r, 0)),
            pl.BlockSpec((3, N), lambda r: (0, 0)),
        ],
        out_specs=[
            pl.BlockSpec((1, N), lambda r: (0, 0)),
            pl.BlockSpec((1, 1), lambda r: (0, 0)),
            pl.BlockSpec((BM, 16), lambda r: (r, 0)),
            pl.BlockSpec((BM, 16), lambda r: (r, 0)),
        ],
        out_shape=[
            jax.ShapeDtypeStruct((1, N), jnp.float32),
            jax.ShapeDtypeStruct((1, 1), jnp.float32),
            jax.ShapeDtypeStruct((N, 16), jnp.int32),
            jax.ShapeDtypeStruct((N, 16), jnp.float32),
        ],
    )(src, dstT)


def _sc_interp_body(c2_hbm, idx_hbm, val_hbm, out_hbm,
                    c2_v, idx_v, val_v, out_v):
    wid = lax.axis_index("s") * SC_NC + lax.axis_index("c")
    pltpu.sync_copy(c2_hbm, c2_v)
    pltpu.sync_copy(idx_hbm.at[pl.ds(wid * (CHUNK * 16), CHUNK * 16)], idx_v)
    pltpu.sync_copy(val_hbm.at[pl.ds(wid * (CHUNK * 16), CHUNK * 16)], val_v)
    iota = lax.iota(jnp.int32, 16)
    for g in range(GROUPS):
        lane_pt = g * 16 + iota
        nacc = jnp.zeros((16,), jnp.float32)
        ax = jnp.zeros((16,), jnp.float32)
        ay = jnp.zeros((16,), jnp.float32)
        az = jnp.zeros((16,), jnp.float32)
        for t in range(K5):
            pos = lane_pt * 16 + t
            iv = plsc.load_gather(idx_v, [pos])
            dv = plsc.load_gather(val_v, [pos])
            w = 1.0 / (dv + 1e-8)
            nacc = nacc + w
            iv3 = iv * 3
            ax = ax + w * plsc.load_gather(c2_v, [iv3])
            ay = ay + w * plsc.load_gather(c2_v, [iv3 + 1])
            az = az + w * plsc.load_gather(c2_v, [iv3 + 2])
        opos = lane_pt * 3
        plsc.store_scatter(out_v, [opos], ax / nacc)
        plsc.store_scatter(out_v, [opos + 1], ay / nacc)
        plsc.store_scatter(out_v, [opos + 2], az / nacc)
    pltpu.sync_copy(out_v, out_hbm.at[pl.ds(wid * (CHUNK * 3), CHUNK * 3)])


def _sc_interp_call(c2_flat, idx_flat, val_flat):
    mesh = plsc.VectorSubcoreMesh(core_axis_name="c", subcore_axis_name="s")
    fn = functools.partial(
        pl.kernel,
        mesh=mesh,
        compiler_params=pltpu.CompilerParams(needs_layout_passes=False),
        out_type=jax.ShapeDtypeStruct((N * 3,), jnp.float32),
        scratch_types=[
            pltpu.VMEM((N * 3,), jnp.float32),
            pltpu.VMEM((CHUNK * 16,), jnp.int32),
            pltpu.VMEM((CHUNK * 16,), jnp.float32),
            pltpu.VMEM((CHUNK * 3,), jnp.float32),
        ],
    )(_sc_interp_body)
    return fn(c2_flat, idx_flat, val_flat)


def kernel(registration_pred, registration_gt, coords):
    flow = registration_pred[0]
    pc1 = coords
    pc2 = coords + registration_gt[0]
    warp = pc1 + flow

    pc1T = pc1.T
    pc2T = pc2.T
    flowT = flow.T

    c2 = _self1_call(pc2, pc2T, pc2)
    _dist2, ch, idxw2, valw2 = _cross_call(warp, pc2T)
    inter = _sc_interp_call(c2.reshape(-1), idxw2.reshape(-1),
                            valw2.reshape(-1)).reshape(N, 3)
    mc, sm = _self2_call(pc1, pc1T, warp, flowT, flow)
    cv = _cv_call(inter, mc)

    chamfer = ch[0, 0]
    curv = cv[0, 0]
    smooth = sm[0, 0] / 8.0

    alpha = 0.02
    total = alpha * chamfer + 0.3 * (alpha * curv) + alpha * smooth
    return jnp.reshape(total, (1,))
